# baseline (speedup 1.0000x reference)
import functools

import jax
import jax.numpy as jnp
from jax.experimental import pallas as pl
from jax.experimental.pallas import tpu as pltpu


def _gcn_layer_kernel(adj_ref, x_ref, w_ref, b_ref, o_ref, s_ref, *, leaky):
    @pl.when(pl.program_id(1) == 0)
    def _():
        s_ref[...] = jnp.dot(
            x_ref[...], w_ref[...], preferred_element_type=jnp.float32
        )

    out = (
        jnp.dot(adj_ref[...], s_ref[...], preferred_element_type=jnp.float32)
        + b_ref[...]
    )
    if leaky:
        out = jnp.where(out >= 0, out, 0.2 * out)
    o_ref[...] = out


def _row_tile(n):
    best = 8
    for bm in range(8, 513, 8):
        if n % bm == 0:
            best = bm
    return best


def _gcn_layer(adj, x, w, b, *, leaky):
    bsz, n, _ = adj.shape
    d = w.shape[1]
    bm = _row_tile(n)
    grid = (bsz, n // bm)

    if x.ndim == 2:
        x_spec = pl.BlockSpec((n, d), lambda bi, mi: (0, 0))
    else:
        x_spec = pl.BlockSpec((None, n, d), lambda bi, mi: (bi, 0, 0))

    return pl.pallas_call(
        functools.partial(_gcn_layer_kernel, leaky=leaky),
        grid=grid,
        in_specs=[
            pl.BlockSpec((None, bm, n), lambda bi, mi: (bi, mi, 0)),
            x_spec,
            pl.BlockSpec((d, d), lambda bi, mi: (0, 0)),
            pl.BlockSpec((1, d), lambda bi, mi: (0, 0)),
        ],
        out_specs=pl.BlockSpec((None, bm, d), lambda bi, mi: (bi, mi, 0)),
        out_shape=jax.ShapeDtypeStruct((bsz, n, d), jnp.float32),
        scratch_shapes=[pltpu.VMEM((n, d), jnp.float32)],
    )(adj, x, w, b)


@jax.jit
def kernel(batch, bx, W1, b1, W2, b2):
    b1 = b1.reshape(1, -1)
    b2 = b2.reshape(1, -1)
    x1 = _gcn_layer(batch, bx, W1, b1, leaky=True)
    out = _gcn_layer(batch, x1, W2, b2, leaky=False)
    return out
